# baseline (speedup 1.0000x reference)
import functools

import jax
import jax.numpy as jnp
from jax import lax
from jax.experimental import pallas as pl
from jax.experimental.pallas import tpu as pltpu
from jax.experimental.pallas import tpu_sc as plsc

N = 16384
D = 64
VOCAB_ROWS = 1000000
LANES = 16
CHUNK = 128
TBLK = 512


def _tc_body(in_ref, out_ref):
    x = in_ref[...]
    y = jnp.transpose(x)
    out_ref[:, 0:D] = y
    out_ref[:, D:2 * D] = jnp.zeros((TBLK, D), jnp.float32)


def _make_relayout():
    grid = (VOCAB_ROWS + TBLK - 1) // TBLK
    return pl.pallas_call(
        _tc_body,
        grid=(grid,),
        in_specs=[pl.BlockSpec((D, TBLK), lambda g: (0, g))],
        out_specs=pl.BlockSpec((TBLK, 2 * D), lambda g: (g, 0)),
        out_shape=jax.ShapeDtypeStruct((VOCAB_ROWS, 2 * D), jnp.float32),
    )


def _make_gather():
    info = plsc.get_sparse_core_info()
    nc, ns = info.num_cores, info.num_subcores
    nw = nc * ns
    b_per_w = N // nw
    n_chunks = b_per_w // CHUNK
    mesh = plsc.VectorSubcoreMesh(core_axis_name="c", subcore_axis_name="s")

    @functools.partial(
        pl.kernel,
        mesh=mesh,
        out_type=jax.ShapeDtypeStruct((N // 2, 2 * D), jnp.float32),
        scratch_types=[
            pltpu.VMEM((b_per_w,), jnp.int32),
            pltpu.VMEM((b_per_w, 2 * D), jnp.float32),
            pltpu.VMEM((b_per_w // 2, 2 * D), jnp.float32),
            pltpu.SemaphoreType.DMA,
        ],
        compiler_params=pltpu.CompilerParams(use_tc_tiling_on_sc=True,
                                             needs_layout_passes=False),
    )
    def emb_kernel(tok_hbm, tw_hbm, pos_hbm, out_hbm,
                   tok_v, rows_v, out_v, sem):
        wid = lax.axis_index("s") * nc + lax.axis_index("c")
        base = pl.multiple_of(wid * b_per_w, b_per_w)
        base2 = pl.multiple_of(wid * (b_per_w // 2), b_per_w // 2)

        pltpu.sync_copy(tok_hbm.at[pl.ds(base, b_per_w)], tok_v)

        copies = [
            pltpu.async_copy(
                tw_hbm.at[tok_v.at[pl.ds(k * CHUNK, CHUNK)]],
                rows_v.at[pl.ds(k * CHUNK, CHUNK)],
                sem)
            for k in range(n_chunks)
        ]
        pltpu.sync_copy(pos_hbm.at[pl.ds(base2, b_per_w // 2)], out_v)
        for c in copies:
            c.wait()

        def compact(p, c):
            for h in range(2):
                for cc in range(D // LANES):
                    osl = pl.ds(h * D + cc * LANES, LANES)
                    ssl = pl.ds(cc * LANES, LANES)
                    out_v[p, osl] = out_v[p, osl] + rows_v[2 * p + h, ssl]
            return c
        lax.fori_loop(0, b_per_w // 2, compact, 0)

        pltpu.sync_copy(out_v, out_hbm.at[pl.ds(base2, b_per_w // 2)])

    return emb_kernel


_relayout = _make_relayout()
_emb = _make_gather()


def kernel(tokens, token_weight, pos_weight):
    tw_wide = _relayout(token_weight.T)
    out = _emb(tokens.astype(jnp.int32),
               tw_wide,
               pos_weight.reshape(-1, 2 * D))
    return out.reshape(N, D)


# speedup vs baseline: 4.8930x; 4.8930x over previous
import functools

import jax
import jax.numpy as jnp
from jax import lax
from jax.experimental import pallas as pl
from jax.experimental.pallas import tpu as pltpu
from jax.experimental.pallas import tpu_sc as plsc

N = 16384
D = 64
LANES = 16
NBUF = 8
GRP = 16


def _make_gather():
    info = plsc.get_sparse_core_info()
    nc, ns = info.num_cores, info.num_subcores
    nw = nc * ns
    b_per_w = N // nw
    mesh = plsc.VectorSubcoreMesh(core_axis_name="c", subcore_axis_name="s")

    @functools.partial(
        pl.kernel,
        mesh=mesh,
        out_type=jax.ShapeDtypeStruct((N // 2, 2 * D), jnp.float32),
        scratch_types=[
            pltpu.VMEM((b_per_w,), jnp.int32),
            pltpu.VMEM((NBUF, D, 2 * D), jnp.float32),
            pltpu.VMEM((b_per_w // 2, 2 * D), jnp.float32),
            [pltpu.SemaphoreType.DMA] * NBUF,
        ],
        compiler_params=pltpu.CompilerParams(use_tc_tiling_on_sc=True,
                                             needs_layout_passes=False),
    )
    def emb_kernel(tok_hbm, twt_hbm, pos_hbm, out_hbm,
                   tok_v, grp_v, out_v, sems):
        wid = lax.axis_index("s") * nc + lax.axis_index("c")
        base = pl.multiple_of(wid * b_per_w, b_per_w)
        base2 = pl.multiple_of(wid * (b_per_w // 2), b_per_w // 2)

        pltpu.sync_copy(tok_hbm.at[pl.ds(base, b_per_w)], tok_v)
        pltpu.sync_copy(pos_hbm.at[pl.ds(base2, b_per_w // 2)], out_v)

        iota = lax.iota(jnp.int32, LANES)

        def fetch(t, buf):
            c128 = pl.multiple_of((t >> 7) * 128, 128)
            return pltpu.async_copy(
                twt_hbm.at[pl.ds(0, D), pl.ds(c128, 128)],
                grp_v.at[buf], sems[buf])

        tvec0 = tok_v[pl.ds(0, GRP)]
        for j in range(NBUF):
            fetch(tvec0[j], j)

        n_groups = b_per_w // GRP

        def body(g, c):
            goff = pl.multiple_of(g * GRP, GRP)
            tcur = tok_v[pl.ds(goff, GRP)]
            gnxt = lax.min(g + 1, n_groups - 1)
            tnxt = tok_v[pl.ds(pl.multiple_of(gnxt * GRP, GRP), GRP)]
            last = g == n_groups - 1
            for j in range(GRP):
                buf = j % NBUF
                pltpu.make_async_copy(
                    twt_hbm.at[pl.ds(0, D), pl.ds(0, 128)],
                    grp_v.at[buf], sems[buf]).wait()
                s = tcur[j] & 127
                svec = jnp.broadcast_to(s, (LANES,))
                bvec = jnp.broadcast_to(jnp.int32(buf), (LANES,))
                p = g * (GRP // 2) + (j >> 1)
                h = (j & 1) * D
                for cc in range(D // LANES):
                    cvec = cc * LANES + iota
                    val = plsc.load_gather(grp_v, [bvec, cvec, svec])
                    osl = pl.ds(h + cc * LANES, LANES)
                    out_v[p, osl] = out_v[p, osl] + val

                if j < GRP - NBUF:
                    fetch(tcur[j + NBUF], buf)
                else:
                    @pl.when(jnp.logical_not(last))
                    def _():
                        fetch(tnxt[j + NBUF - GRP], buf)
            return c

        lax.fori_loop(0, n_groups, body, 0)

        pltpu.sync_copy(out_v, out_hbm.at[pl.ds(base2, b_per_w // 2)])

    return emb_kernel


_emb = _make_gather()


def kernel(tokens, token_weight, pos_weight):
    out = _emb(tokens.astype(jnp.int32),
               token_weight.T,
               pos_weight.reshape(-1, 2 * D))
    return out.reshape(N, D)


# speedup vs baseline: 6.1121x; 1.2491x over previous
import functools

import jax
import jax.numpy as jnp
from jax import lax
from jax.experimental import pallas as pl
from jax.experimental.pallas import tpu as pltpu
from jax.experimental.pallas import tpu_sc as plsc

N = 16384
D = 64
LANES = 16
NBUF = 8
GRP = 16


def _make_gather():
    info = plsc.get_sparse_core_info()
    nc, ns = info.num_cores, info.num_subcores
    nw = nc * ns
    b_per_w = N // nw
    mesh = plsc.VectorSubcoreMesh(core_axis_name="c", subcore_axis_name="s")

    @functools.partial(
        pl.kernel,
        mesh=mesh,
        out_type=jax.ShapeDtypeStruct((D, N), jnp.float32),
        scratch_types=[
            pltpu.VMEM((b_per_w,), jnp.int32),
            pltpu.VMEM((NBUF, D, 2 * D), jnp.float32),
            pltpu.VMEM((D, b_per_w), jnp.float32),
            [pltpu.SemaphoreType.DMA] * NBUF,
        ],
        compiler_params=pltpu.CompilerParams(use_tc_tiling_on_sc=True,
                                             needs_layout_passes=False),
    )
    def emb_kernel(tok_hbm, twt_hbm, post_hbm, out_hbm,
                   tok_v, grp_v, out_v, sems):
        wid = lax.axis_index("s") * nc + lax.axis_index("c")
        base = pl.multiple_of(wid * b_per_w, b_per_w)

        pltpu.sync_copy(tok_hbm.at[pl.ds(base, b_per_w)], tok_v)
        pltpu.sync_copy(post_hbm.at[pl.ds(0, D), pl.ds(base, b_per_w)], out_v)

        iota = lax.iota(jnp.int32, LANES)

        def fetch(t, buf):
            c128 = pl.multiple_of((t >> 7) * 128, 128)
            return pltpu.async_copy(
                twt_hbm.at[pl.ds(0, D), pl.ds(c128, 128)],
                grp_v.at[buf], sems[buf])

        tvec0 = tok_v[pl.ds(0, GRP)]
        for j in range(NBUF):
            fetch(tvec0[j], j)

        n_groups = b_per_w // GRP

        def body(g, c):
            goff = pl.multiple_of(g * GRP, GRP)
            tcur = tok_v[pl.ds(goff, GRP)]
            gnxt = lax.min(g + 1, n_groups - 1)
            tnxt = tok_v[pl.ds(pl.multiple_of(gnxt * GRP, GRP), GRP)]
            last = g == n_groups - 1
            for j in range(GRP):
                buf = j % NBUF
                pltpu.make_async_copy(
                    twt_hbm.at[pl.ds(0, D), pl.ds(0, 128)],
                    grp_v.at[buf], sems[buf]).wait()
                s = tcur[j] & 127
                svec = jnp.broadcast_to(s, (LANES,))
                bvec = jnp.broadcast_to(jnp.int32(buf), (LANES,))
                rvec = jnp.broadcast_to(goff + j, (LANES,))
                for cc in range(D // LANES):
                    cvec = cc * LANES + iota
                    val = plsc.load_gather(grp_v, [bvec, cvec, svec])
                    plsc.addupdate_scatter(out_v, [cvec, rvec], val)

                if j < GRP - NBUF:
                    fetch(tcur[j + NBUF], buf)
                else:
                    @pl.when(jnp.logical_not(last))
                    def _():
                        fetch(tnxt[j + NBUF - GRP], buf)
            return c

        lax.fori_loop(0, n_groups, body, 0)

        pltpu.sync_copy(out_v, out_hbm.at[pl.ds(0, D), pl.ds(base, b_per_w)])

    return emb_kernel


_emb = _make_gather()


def kernel(tokens, token_weight, pos_weight):
    out_t = _emb(tokens.astype(jnp.int32),
                 token_weight.T,
                 pos_weight.T)
    return out_t.T


# speedup vs baseline: 6.1307x; 1.0031x over previous
import functools

import jax
import jax.numpy as jnp
from jax import lax
from jax.experimental import pallas as pl
from jax.experimental.pallas import tpu as pltpu
from jax.experimental.pallas import tpu_sc as plsc

N = 16384
D = 64
LANES = 16
NBUF = 8
GRP = 16


def _make_gather():
    info = plsc.get_sparse_core_info()
    nc, ns = info.num_cores, info.num_subcores
    nw = nc * ns
    b_per_w = N // nw
    mesh = plsc.VectorSubcoreMesh(core_axis_name="c", subcore_axis_name="s")

    @functools.partial(
        pl.kernel,
        mesh=mesh,
        out_type=jax.ShapeDtypeStruct((D, N), jnp.float32),
        scratch_types=[
            pltpu.VMEM((b_per_w,), jnp.int32),
            pltpu.VMEM((NBUF, D, 2 * D), jnp.float32),
            pltpu.VMEM((D, b_per_w), jnp.float32),
            [pltpu.SemaphoreType.DMA] * NBUF,
        ],
        compiler_params=pltpu.CompilerParams(use_tc_tiling_on_sc=True,
                                             needs_layout_passes=False),
    )
    def emb_kernel(tok_hbm, twt_hbm, post_hbm, out_hbm,
                   tok_v, grp_v, out_v, sems):
        wid = lax.axis_index("s") * nc + lax.axis_index("c")
        base = pl.multiple_of(wid * b_per_w, b_per_w)

        pltpu.sync_copy(tok_hbm.at[pl.ds(base, b_per_w)], tok_v)

        iota = lax.iota(jnp.int32, LANES)

        def fetch(t, buf):
            c128 = pl.multiple_of((t >> 7) * 128, 128)
            return pltpu.async_copy(
                twt_hbm.at[pl.ds(0, D), pl.ds(c128, 128)],
                grp_v.at[buf], sems[buf])

        tvec0 = tok_v[pl.ds(0, GRP)]
        for j in range(NBUF):
            fetch(tvec0[j], j)
        pltpu.sync_copy(post_hbm.at[pl.ds(0, D), pl.ds(base, b_per_w)], out_v)

        n_groups = b_per_w // GRP

        def body(g, c):
            goff = pl.multiple_of(g * GRP, GRP)
            tcur = tok_v[pl.ds(goff, GRP)]
            gnxt = lax.min(g + 1, n_groups - 1)
            tnxt = tok_v[pl.ds(pl.multiple_of(gnxt * GRP, GRP), GRP)]
            last = g == n_groups - 1
            for j in range(GRP):
                buf = j % NBUF
                pltpu.make_async_copy(
                    twt_hbm.at[pl.ds(0, D), pl.ds(0, 128)],
                    grp_v.at[buf], sems[buf]).wait()
                s = tcur[j] & 127
                svec = jnp.broadcast_to(s, (LANES,))
                bvec = jnp.broadcast_to(jnp.int32(buf), (LANES,))
                rvec = jnp.broadcast_to(goff + j, (LANES,))
                for cc in range(D // LANES):
                    cvec = cc * LANES + iota
                    val = plsc.load_gather(grp_v, [bvec, cvec, svec])
                    plsc.addupdate_scatter(out_v, [cvec, rvec], val)

                if j < GRP - NBUF:
                    fetch(tcur[j + NBUF], buf)
                else:
                    @pl.when(jnp.logical_not(last))
                    def _():
                        fetch(tnxt[j + NBUF - GRP], buf)
            return c

        lax.fori_loop(0, n_groups, body, 0)

        pltpu.sync_copy(out_v, out_hbm.at[pl.ds(0, D), pl.ds(base, b_per_w)])

    return emb_kernel


_emb = _make_gather()


def kernel(tokens, token_weight, pos_weight):
    out_t = _emb(tokens.astype(jnp.int32),
                 token_weight.T,
                 pos_weight.T)
    return out_t.T
